# baseline (speedup 1.0000x reference)
import functools

import jax
import jax.numpy as jnp
from jax import lax
from jax.experimental import pallas as pl
from jax.experimental.pallas import tpu as pltpu

VOCAB = 1000
VOCAB_PAD = 1024
D_MODEL = 768
B_TOTAL = 1024 * 200
BLK = 512
N_BLK = B_TOTAL // BLK


def _tc_body(idx_ref, table_ref, out_ref):
    idx = idx_ref[...]
    iota = lax.broadcasted_iota(jnp.int32, (BLK, VOCAB_PAD), 1)
    oh = (idx == iota).astype(jnp.bfloat16)
    out_ref[...] = jnp.dot(
        oh, table_ref[...], preferred_element_type=jnp.float32
    )


@jax.jit
def _tc_lookup(idx2d, table_bf):
    return pl.pallas_call(
        _tc_body,
        grid=(N_BLK,),
        in_specs=[
            pl.BlockSpec((BLK, 1), lambda i: (i, 0)),
            pl.BlockSpec((VOCAB_PAD, D_MODEL), lambda i: (0, 0)),
        ],
        out_specs=pl.BlockSpec((BLK, D_MODEL), lambda i: (i, 0)),
        out_shape=jax.ShapeDtypeStruct((B_TOTAL, D_MODEL), jnp.float32),
    )(idx2d, table_bf)


def kernel(token_ids, embeddings):
    idx = token_ids.reshape(-1, 1).astype(jnp.int32)
    table_bf = jnp.pad(
        embeddings, ((0, VOCAB_PAD - VOCAB), (0, 0))
    ).astype(jnp.bfloat16)
    out = _tc_lookup(idx, table_bf)
    return out.reshape(token_ids.shape + (D_MODEL,))
